# baseline (speedup 1.0000x reference)
import jax
import jax.numpy as jnp
from jax.experimental import pallas as pl
from jax.experimental.pallas import tpu as pltpu

_DIM = 256
_NQ = 4
_K = 1024
_B = 8192
_BB = 1024
_ILANES = 128


def _rvq_block(x_ref, cb_ref, q_ref, idx_ref, wcat_s):
    @pl.when(pl.program_id(0) == 0)
    def _prep():
        w = cb_ref[...]
        w1 = w.astype(jnp.bfloat16)
        r1 = w - w1.astype(jnp.float32)
        w2 = r1.astype(jnp.bfloat16)
        w3 = (r1 - w2.astype(jnp.float32)).astype(jnp.bfloat16)
        wcat_s[:, :, 0:_DIM] = w1
        wcat_s[:, :, _DIM:2 * _DIM] = w2
        wcat_s[:, :, 2 * _DIM:3 * _DIM] = w3

    r = x_ref[...]
    ks = jax.lax.broadcasted_iota(jnp.int32, (_BB, _K), 1)
    for i in range(_NQ):
        w = cb_ref[i]
        x2 = jnp.sum(r * r, axis=-1, keepdims=True)
        w2 = jnp.sum(w * w, axis=-1)[None, :]
        s = jax.lax.dot_general(
            r, w, (((1,), (1,)), ((), ())),
            preferred_element_type=jnp.float32,
        )
        dist = (x2 + w2) - 2.0 * s
        idx = jnp.argmin(dist, axis=-1).astype(jnp.int32)
        oh = (ks == idx[:, None]).astype(jnp.bfloat16)
        q3 = jax.lax.dot_general(
            oh, wcat_s[i], (((1,), (0,)), ((), ())),
            preferred_element_type=jnp.float32)
        g1 = q3[:, 0:_DIM]
        g2 = q3[:, _DIM:2 * _DIM]
        g3 = q3[:, 2 * _DIM:3 * _DIM]
        q = g1 + (g2 + g3)
        q_ref[:, i, :] = r + (q - r)
        idx_ref[:, i] = idx
        r = r - q


def kernel(x, codebooks):
    grid = (_B // _BB,)
    quantized, idx_pad = pl.pallas_call(
        _rvq_block,
        grid=grid,
        in_specs=[
            pl.BlockSpec((_BB, _DIM), lambda b: (b, 0)),
            pl.BlockSpec((_NQ, _K, _DIM), lambda b: (0, 0, 0)),
        ],
        out_specs=[
            pl.BlockSpec((_BB, _NQ, _DIM), lambda b: (b, 0, 0)),
            pl.BlockSpec((_BB, _ILANES), lambda b: (b, 0)),
        ],
        out_shape=[
            jax.ShapeDtypeStruct((_B, _NQ, _DIM), jnp.float32),
            jax.ShapeDtypeStruct((_B, _ILANES), jnp.int32),
        ],
        scratch_shapes=[
            pltpu.VMEM((_NQ, _K, 3 * _DIM), jnp.bfloat16),
        ],
    )(x, codebooks)
    indices = idx_pad[:, :_NQ]
    loss = jnp.zeros((), dtype=jnp.float32)
    return quantized, indices, loss


# speedup vs baseline: 1.0280x; 1.0280x over previous
import jax
import jax.numpy as jnp
from jax.experimental import pallas as pl
from jax.experimental.pallas import tpu as pltpu

_DIM = 256
_NQ = 4
_K = 1024
_B = 8192
_BB = 1024
_ILANES = 128


def _rvq_block(x_ref, cb_ref, q_ref, idx_ref, w1_s, w2_s, w3_s):
    @pl.when(pl.program_id(0) == 0)
    def _prep():
        w = cb_ref[...]
        w1 = w.astype(jnp.bfloat16)
        r1 = w - w1.astype(jnp.float32)
        w2 = r1.astype(jnp.bfloat16)
        w3 = (r1 - w2.astype(jnp.float32)).astype(jnp.bfloat16)
        w1_s[...] = w1
        w2_s[...] = w2
        w3_s[...] = w3

    r = x_ref[...]
    ks = jax.lax.broadcasted_iota(jnp.int32, (_BB, _K), 1)
    for i in range(_NQ):
        w = cb_ref[i]
        x2 = jnp.sum(r * r, axis=-1, keepdims=True)
        w2 = jnp.sum(w * w, axis=-1)[None, :]
        s = jax.lax.dot_general(
            r, w, (((1,), (1,)), ((), ())),
            preferred_element_type=jnp.float32,
        )
        dist = (x2 + w2) - 2.0 * s
        m = jnp.min(dist, axis=-1, keepdims=True)
        idx = jnp.min(jnp.where(dist == m, ks, _K), axis=-1)
        oh = (ks == idx[:, None]).astype(jnp.bfloat16)
        g1 = jax.lax.dot_general(
            oh, w1_s[i], (((1,), (0,)), ((), ())),
            preferred_element_type=jnp.float32)
        g2 = jax.lax.dot_general(
            oh, w2_s[i], (((1,), (0,)), ((), ())),
            preferred_element_type=jnp.float32)
        g3 = jax.lax.dot_general(
            oh, w3_s[i], (((1,), (0,)), ((), ())),
            preferred_element_type=jnp.float32)
        q = g1 + (g2 + g3)
        q_ref[:, i, :] = r + (q - r)
        idx_ref[:, i] = idx
        r = r - q


def kernel(x, codebooks):
    grid = (_B // _BB,)
    quantized, idx_pad = pl.pallas_call(
        _rvq_block,
        grid=grid,
        in_specs=[
            pl.BlockSpec((_BB, _DIM), lambda b: (b, 0)),
            pl.BlockSpec((_NQ, _K, _DIM), lambda b: (0, 0, 0)),
        ],
        out_specs=[
            pl.BlockSpec((_BB, _NQ, _DIM), lambda b: (b, 0, 0)),
            pl.BlockSpec((_BB, _ILANES), lambda b: (b, 0)),
        ],
        out_shape=[
            jax.ShapeDtypeStruct((_B, _NQ, _DIM), jnp.float32),
            jax.ShapeDtypeStruct((_B, _ILANES), jnp.int32),
        ],
        scratch_shapes=[
            pltpu.VMEM((_NQ, _K, _DIM), jnp.bfloat16),
            pltpu.VMEM((_NQ, _K, _DIM), jnp.bfloat16),
            pltpu.VMEM((_NQ, _K, _DIM), jnp.bfloat16),
        ],
    )(x, codebooks)
    indices = idx_pad[:, :_NQ]
    loss = jnp.zeros((), dtype=jnp.float32)
    return quantized, indices, loss
